# baseline (speedup 1.0000x reference)
import functools

import jax
import jax.numpy as jnp
from jax import lax
from jax.experimental import pallas as pl
from jax.experimental.pallas import tpu as pltpu
from jax.experimental.pallas import tpu_sc as plsc

NTOKEN = 100000
EMB_DIM = 64
BATCH = 4096
HIST = 50
LANES = 16
NGRP = BATCH // LANES

_info = plsc.get_sparse_core_info()
NC = _info.num_cores
NS = _info.num_subcores
NW = NC * NS
NPASS = EMB_DIM // NW

_mesh = plsc.VectorSubcoreMesh(core_axis_name="c", subcore_axis_name="s")


@functools.partial(
    pl.kernel,
    mesh=_mesh,
    out_type=jax.ShapeDtypeStruct((HIST, EMB_DIM, BATCH), jnp.float32),
    scratch_types=[
        pltpu.VMEM((NTOKEN + 1,), jnp.float32),
        pltpu.VMEM((BATCH,), jnp.int32),
        pltpu.VMEM((BATCH,), jnp.int32),
        pltpu.VMEM((BATCH,), jnp.float32),
        pltpu.VMEM((BATCH,), jnp.float32),
        pltpu.SemaphoreType.DMA,
        pltpu.SemaphoreType.DMA,
        pltpu.SemaphoreType.DMA,
        pltpu.SemaphoreType.DMA,
    ],
    compiler_params=pltpu.CompilerParams(use_tc_tiling_on_sc=True,
                                         needs_layout_passes=False),
)
def _tgather_kernel(tableT_hbm, xT_hbm, out_hbm, row_v, x0, x1, s0, s1,
                    lx0, lx1, ss0, ss1):
    wid = lax.axis_index("s") * NC + lax.axis_index("c")
    xv, sv, lx, ss = [x0, x1], [s0, s1], [lx0, lx1], [ss0, ss1]

    def gather_slab(sub):
        W = 10
        idxs = [None] * NGRP
        vals = [None] * NGRP
        for i in range(NGRP + 2 * W):
            if i < NGRP:
                idxs[i] = xv[sub][pl.ds(i * LANES, LANES)]
            j = i - W
            if 0 <= j < NGRP:
                vals[j] = plsc.load_gather(row_v, [idxs[j]])
                idxs[j] = None
            k = i - 2 * W
            if 0 <= k < NGRP:
                sv[sub][pl.ds(k * LANES, LANES)] = vals[k]
                vals[k] = None

    for p in range(NPASS):
        e = wid + NW * p
        pltpu.async_copy(xT_hbm.at[0], xv[0], lx[0])
        pltpu.async_copy(xT_hbm.at[1], xv[1], lx[1])
        pltpu.sync_copy(tableT_hbm.at[e], row_v)

        def pair(g, carry):
            for sub in range(2):
                l = 2 * g + sub
                @pl.when(g >= 1)
                def _wait_store():
                    pltpu.make_async_copy(sv[sub], out_hbm.at[l, e], ss[sub]).wait()

                pltpu.make_async_copy(xT_hbm.at[l], xv[sub], lx[sub]).wait()

                gather_slab(sub)
                pltpu.async_copy(sv[sub], out_hbm.at[l, e], ss[sub])

                @pl.when(g < HIST // 2 - 1)
                def _prefetch():
                    pltpu.async_copy(xT_hbm.at[l + 2], xv[sub], lx[sub])
            return carry

        lax.fori_loop(0, HIST // 2, pair, 0)
        for sub in range(2):
            pltpu.make_async_copy(sv[sub], out_hbm.at[0, e], ss[sub]).wait()


def kernel(x, table):
    out_t = _tgather_kernel(table.T, x.T.astype(jnp.int32))
    return out_t.transpose(2, 0, 1)


# speedup vs baseline: 1.5368x; 1.5368x over previous
import functools

import jax
import jax.numpy as jnp
from jax import lax
from jax.experimental import pallas as pl
from jax.experimental.pallas import tpu as pltpu
from jax.experimental.pallas import tpu_sc as plsc

NTOKEN = 100000
EMB_DIM = 64
BATCH = 4096
HIST = 50
LANES = 16
NGRP = BATCH // LANES

_info = plsc.get_sparse_core_info()
NC = _info.num_cores
NS = _info.num_subcores
NW = NC * NS
NPASS = EMB_DIM // NW

_mesh = plsc.VectorSubcoreMesh(core_axis_name="c", subcore_axis_name="s")


@functools.partial(
    pl.kernel,
    mesh=_mesh,
    out_type=jax.ShapeDtypeStruct((HIST, EMB_DIM, BATCH), jnp.float32),
    scratch_types=[
        pltpu.VMEM((NTOKEN + 1,), jnp.float32),
        pltpu.VMEM((BATCH,), jnp.int32),
        pltpu.VMEM((BATCH,), jnp.int32),
        pltpu.VMEM((BATCH,), jnp.float32),
        pltpu.VMEM((BATCH,), jnp.float32),
        pltpu.VMEM_SHARED((HIST, BATCH), jnp.int32),
        pltpu.SemaphoreType.DMA,
        pltpu.SemaphoreType.DMA,
        pltpu.SemaphoreType.DMA,
        pltpu.SemaphoreType.DMA,
    ],
    compiler_params=pltpu.CompilerParams(use_tc_tiling_on_sc=True,
                                         needs_layout_passes=False),
)
def _tgather_kernel(tableT_hbm, xT_hbm, out_hbm, row_v, x0, x1, s0, s1,
                    xsh, lx0, lx1, ss0, ss1):
    wid = lax.axis_index("s") * NC + lax.axis_index("c")
    sid = lax.axis_index("s")
    xv, sv, lx, ss = [x0, x1], [s0, s1], [lx0, lx1], [ss0, ss1]

    for r in range(4):
        li = sid * 4 + r

        @pl.when(li < HIST)
        def _stage():
            pltpu.sync_copy(xT_hbm.at[li], xsh.at[li])

    plsc.subcore_barrier()

    def gather_slab(sub):
        W = 10
        idxs = [None] * NGRP
        vals = [None] * NGRP
        for i in range(NGRP + 2 * W):
            if i < NGRP:
                idxs[i] = xv[sub][pl.ds(i * LANES, LANES)]
            j = i - W
            if 0 <= j < NGRP:
                vals[j] = plsc.load_gather(row_v, [idxs[j]])
                idxs[j] = None
            k = i - 2 * W
            if 0 <= k < NGRP:
                sv[sub][pl.ds(k * LANES, LANES)] = vals[k]
                vals[k] = None

    for p in range(NPASS):
        e = wid + NW * p
        pltpu.async_copy(xsh.at[0], xv[0], lx[0])
        pltpu.async_copy(xsh.at[1], xv[1], lx[1])
        pltpu.sync_copy(tableT_hbm.at[e], row_v)

        def pair(g, carry):
            for sub in range(2):
                l = 2 * g + sub
                @pl.when(g >= 1)
                def _wait_store():
                    pltpu.make_async_copy(sv[sub], out_hbm.at[l, e], ss[sub]).wait()

                pltpu.make_async_copy(xsh.at[l], xv[sub], lx[sub]).wait()

                gather_slab(sub)
                pltpu.async_copy(sv[sub], out_hbm.at[l, e], ss[sub])

                @pl.when(g < HIST // 2 - 1)
                def _prefetch():
                    pltpu.async_copy(xsh.at[l + 2], xv[sub], lx[sub])
            return carry

        lax.fori_loop(0, HIST // 2, pair, 0)
        for sub in range(2):
            pltpu.make_async_copy(sv[sub], out_hbm.at[0, e], ss[sub]).wait()


def kernel(x, table):
    out_t = _tgather_kernel(table.T, x.T.astype(jnp.int32))
    return out_t.transpose(2, 0, 1)


# speedup vs baseline: 1.5392x; 1.0015x over previous
import functools

import jax
import jax.numpy as jnp
from jax import lax
from jax.experimental import pallas as pl
from jax.experimental.pallas import tpu as pltpu
from jax.experimental.pallas import tpu_sc as plsc

NTOKEN = 100000
EMB_DIM = 64
BATCH = 4096
HIST = 50
LANES = 16
NGRP = BATCH // LANES

_info = plsc.get_sparse_core_info()
NC = _info.num_cores
NS = _info.num_subcores
NW = NC * NS
NPASS = EMB_DIM // NW

_mesh = plsc.VectorSubcoreMesh(core_axis_name="c", subcore_axis_name="s")


@functools.partial(
    pl.kernel,
    mesh=_mesh,
    out_type=jax.ShapeDtypeStruct((HIST, EMB_DIM, BATCH), jnp.float32),
    scratch_types=[
        pltpu.VMEM((NTOKEN + 1,), jnp.float32),
        pltpu.VMEM((BATCH,), jnp.int32),
        pltpu.VMEM((BATCH,), jnp.int32),
        pltpu.VMEM((BATCH,), jnp.float32),
        pltpu.VMEM((BATCH,), jnp.float32),
        pltpu.VMEM_SHARED((56, BATCH), jnp.int32),
        pltpu.SemaphoreType.DMA,
        pltpu.SemaphoreType.DMA,
        pltpu.SemaphoreType.DMA,
        pltpu.SemaphoreType.DMA,
    ],
    compiler_params=pltpu.CompilerParams(use_tc_tiling_on_sc=True,
                                         needs_layout_passes=False),
)
def _tgather_kernel(tableT_hbm, xT_hbm, out_hbm, row_v, x0, x1, s0, s1,
                    xsh, lx0, lx1, ss0, ss1):
    wid = lax.axis_index("s") * NC + lax.axis_index("c")
    sid = lax.axis_index("s")
    xv, sv, lx, ss = [x0, x1], [s0, s1], [lx0, lx1], [ss0, ss1]

    for r in range(4):
        li = sid * 4 + r

        @pl.when(li < HIST)
        def _stage():
            pltpu.sync_copy(xT_hbm.at[li], xsh.at[li])

    plsc.subcore_barrier()

    def gather_slab(sub):
        W = 10
        idxs = [None] * NGRP
        vals = [None] * NGRP
        for i in range(NGRP + 2 * W):
            if i < NGRP:
                idxs[i] = xv[sub][pl.ds(i * LANES, LANES)]
            j = i - W
            if 0 <= j < NGRP:
                vals[j] = plsc.load_gather(row_v, [idxs[j]])
                idxs[j] = None
            k = i - 2 * W
            if 0 <= k < NGRP:
                sv[sub][pl.ds(k * LANES, LANES)] = vals[k]
                vals[k] = None

    for p in range(NPASS):
        e = wid + NW * p
        pltpu.async_copy(xsh.at[0], xv[0], lx[0])
        pltpu.async_copy(xsh.at[1], xv[1], lx[1])
        pltpu.sync_copy(tableT_hbm.at[e], row_v)

        def pair(g, carry):
            for sub in range(2):
                l = 2 * g + sub
                @pl.when(g >= 1)
                def _wait_store():
                    pltpu.make_async_copy(sv[sub], out_hbm.at[l, e], ss[sub]).wait()

                pltpu.make_async_copy(xsh.at[l], xv[sub], lx[sub]).wait()

                gather_slab(sub)
                pltpu.async_copy(sv[sub], out_hbm.at[l, e], ss[sub])

                @pl.when(g < HIST // 2 - 1)
                def _prefetch():
                    pltpu.async_copy(xsh.at[l + 2], xv[sub], lx[sub])
            return carry

        lax.fori_loop(0, HIST // 2, pair, 0)
        for sub in range(2):
            pltpu.make_async_copy(sv[sub], out_hbm.at[0, e], ss[sub]).wait()


def kernel(x, table):
    out_t = _tgather_kernel(table.T, x.T.astype(jnp.int32))
    return out_t.transpose(2, 0, 1)
